# Initial kernel scaffold; baseline (speedup 1.0000x reference)
import jax
import jax.numpy as jnp
from jax.experimental import pallas as pl


def kernel(x, gate_w, w1, b1, w2, b2, w3, b3, ln_g, ln_b):
    raise NotImplementedError("write your pallas kernel here")


# speedup vs baseline: 4.1690x; 4.1690x over previous
import functools

import jax
import jax.numpy as jnp
from jax.experimental import pallas as pl
from jax.experimental.pallas import tpu as pltpu

def _gelu_exact(h):
    return 0.5 * h * (1.0 + jax.lax.erf(h * 0.7071067811865476))


E = 8
TOP_K = 2
D = 768
H = 768
O = 768
N_TOK = 4096
TB = 512
NTB = N_TOK // TB


def _moe_kernel(x_ref, gate_w_ref, w1_ref, b1_ref, w2_ref, b2_ref,
                w3_ref, b3_ref, ln_g_ref, ln_b_ref,
                out_ref, aux_ref,
                acc_ref, gates_ref):
    e = pl.program_id(0)
    tb = pl.program_id(1)

    @pl.when(jnp.logical_and(e == 0, tb == 0))
    def _router():
        x_all = x_ref[...]
        logits = jnp.dot(x_all, gate_w_ref[...],
                         preferred_element_type=jnp.float32)
        lane = jax.lax.broadcasted_iota(jnp.int32, (N_TOK, E), 1)
        v1 = jnp.max(logits, axis=1, keepdims=True)
        i1 = jnp.argmax(logits, axis=1)[:, None]
        masked = jnp.where(lane == i1, -jnp.inf, logits)
        v2 = jnp.max(masked, axis=1, keepdims=True)
        i2 = jnp.argmax(masked, axis=1)[:, None]
        g1 = 1.0 / (1.0 + jnp.exp(v2 - v1))
        g2 = 1.0 - g1
        gates = jnp.where(lane == i1, g1, 0.0) + jnp.where(lane == i2, g2, 0.0)
        gates_ref[...] = gates
        usage = jnp.sum(gates, axis=0, keepdims=True) / N_TOK
        uniform = 1.0 / E
        aux = jnp.sum(uniform * (jnp.log(uniform) - jnp.log(usage)),
                      axis=1, keepdims=True)
        aux_ref[...] = aux

    xb = x_ref[pl.ds(tb * TB, TB), :].astype(jnp.bfloat16)
    w1 = w1_ref[0].astype(jnp.bfloat16)
    h = jnp.dot(xb, w1, preferred_element_type=jnp.float32) + b1_ref[0]
    h = _gelu_exact(h)
    h = jnp.dot(h.astype(jnp.bfloat16), w2_ref[0].astype(jnp.bfloat16),
                preferred_element_type=jnp.float32) + b2_ref[0]
    h = _gelu_exact(h)
    y = jnp.dot(h.astype(jnp.bfloat16), w3_ref[0].astype(jnp.bfloat16),
                preferred_element_type=jnp.float32) + b3_ref[0]

    lane_e = jax.lax.broadcasted_iota(jnp.int32, (TB, E), 1)
    gblk = gates_ref[pl.ds(tb * TB, TB), :]
    ge = jnp.sum(jnp.where(lane_e == e, gblk, 0.0), axis=1, keepdims=True)

    prev = jnp.where(e == 0, 0.0, acc_ref[pl.ds(tb * TB, TB), :])
    a = prev + ge * y
    acc_ref[pl.ds(tb * TB, TB), :] = a

    @pl.when(e == E - 1)
    def _finalize():
        mu = jnp.mean(a, axis=1, keepdims=True)
        var = jnp.mean((a - mu) ** 2, axis=1, keepdims=True)
        out_ref[...] = ((a - mu) * jax.lax.rsqrt(var + 1e-5)
                        * ln_g_ref[...] + ln_b_ref[...])


@jax.jit
def kernel(x, gate_w, w1, b1, w2, b2, w3, b3, ln_g, ln_b):
    b, s, d = x.shape
    x2 = x.reshape(b * s, d)
    out2, aux = pl.pallas_call(
        _moe_kernel,
        grid=(E, NTB),
        in_specs=[
            pl.BlockSpec((N_TOK, D), lambda e, tb: (0, 0)),
            pl.BlockSpec((D, E), lambda e, tb: (0, 0)),
            pl.BlockSpec((1, D, H), lambda e, tb: (e, 0, 0)),
            pl.BlockSpec((1, 1, H), lambda e, tb: (e, 0, 0)),
            pl.BlockSpec((1, H, H), lambda e, tb: (e, 0, 0)),
            pl.BlockSpec((1, 1, H), lambda e, tb: (e, 0, 0)),
            pl.BlockSpec((1, H, O), lambda e, tb: (e, 0, 0)),
            pl.BlockSpec((1, 1, O), lambda e, tb: (e, 0, 0)),
            pl.BlockSpec((1, O), lambda e, tb: (0, 0)),
            pl.BlockSpec((1, O), lambda e, tb: (0, 0)),
        ],
        out_specs=[
            pl.BlockSpec((TB, O), lambda e, tb: (tb, 0)),
            pl.BlockSpec((1, 1), lambda e, tb: (0, 0)),
        ],
        out_shape=[
            jax.ShapeDtypeStruct((N_TOK, O), jnp.float32),
            jax.ShapeDtypeStruct((1, 1), jnp.float32),
        ],
        scratch_shapes=[
            pltpu.VMEM((N_TOK, O), jnp.float32),
            pltpu.VMEM((N_TOK, E), jnp.float32),
        ],
    )(x2, gate_w, w1, b1.reshape(E, 1, H), w2, b2.reshape(E, 1, H),
      w3, b3.reshape(E, 1, O),
      ln_g.reshape(1, O), ln_b.reshape(1, O))
    return out2.reshape(b, s, O), aux[0, 0]


# speedup vs baseline: 4.4875x; 1.0764x over previous
import functools

import jax
import jax.numpy as jnp
from jax import lax
from jax.experimental import pallas as pl
from jax.experimental.pallas import tpu as pltpu
from jax.experimental.pallas import tpu_sc as plsc

E = 8
TOP_K = 2
D = 768
H = 768
O = 768
N_TOK = 4096
TB = 512
NTB = N_TOK // TB
NB = N_TOK * TOP_K // TB + E
NROWS = NB * TB

NW = 32
TPW = N_TOK // NW
CH = 64


def _gelu_exact(h):
    return 0.5 * h * (1.0 + jax.lax.erf(h * 0.7071067811865476))


def _router_kernel(x_ref, gw_ref,
                   e1_ref, e2_ref, k1_ref, k2_ref, g1_ref, g2_ref,
                   counts_ref, aux_ref,
                   cnt_s, use_s):
    tb = pl.program_id(0)

    @pl.when(tb == 0)
    def _init():
        cnt_s[...] = jnp.zeros((1, E), jnp.float32)
        use_s[...] = jnp.zeros((1, E), jnp.float32)

    xb = x_ref[...]
    logits = jnp.dot(xb, gw_ref[...], preferred_element_type=jnp.float32)
    lane = jax.lax.broadcasted_iota(jnp.int32, (TB, E), 1)
    v1 = jnp.max(logits, axis=1, keepdims=True)
    i1 = jnp.argmax(logits, axis=1)[:, None]
    masked = jnp.where(lane == i1, -jnp.inf, logits)
    v2 = jnp.max(masked, axis=1, keepdims=True)
    i2 = jnp.argmax(masked, axis=1)[:, None]
    g1 = 1.0 / (1.0 + jnp.exp(v2 - v1))
    g2 = 1.0 - g1

    oh1 = (lane == i1).astype(jnp.float32)
    oh2 = (lane == i2).astype(jnp.float32)
    ohs = oh1 + oh2
    row = jax.lax.broadcasted_iota(jnp.int32, (TB, TB), 0)
    col = jax.lax.broadcasted_iota(jnp.int32, (TB, TB), 1)
    strict_lt = (col < row).astype(jnp.bfloat16)
    excl = jnp.dot(strict_lt, ohs.astype(jnp.bfloat16),
                   preferred_element_type=jnp.float32)
    cnt = cnt_s[...]
    rank1 = jnp.sum(oh1 * (cnt + excl), axis=1, keepdims=True)
    rank2 = jnp.sum(oh2 * (cnt + excl + oh1), axis=1, keepdims=True)
    cnt_s[...] = cnt + jnp.sum(ohs, axis=0, keepdims=True)
    use_s[...] = use_s[...] + jnp.sum(oh1 * g1 + oh2 * g2, axis=0,
                                      keepdims=True)

    e1_ref[...] = i1
    e2_ref[...] = i2
    k1_ref[...] = rank1.astype(jnp.int32)
    k2_ref[...] = rank2.astype(jnp.int32)
    g1_ref[...] = g1
    g2_ref[...] = g2

    @pl.when(tb == NTB - 1)
    def _fin():
        counts_ref[...] = cnt_s[...]
        usage = use_s[...] / N_TOK
        uniform = 1.0 / E
        aux_ref[...] = jnp.sum(uniform * (jnp.log(uniform) - jnp.log(usage)),
                               axis=1, keepdims=True)


def _router(x2, gate_w):
    return pl.pallas_call(
        _router_kernel,
        grid=(NTB,),
        in_specs=[
            pl.BlockSpec((TB, D), lambda tb: (tb, 0)),
            pl.BlockSpec((D, E), lambda tb: (0, 0)),
        ],
        out_specs=[pl.BlockSpec((TB, 1), lambda tb: (tb, 0))] * 6 + [
            pl.BlockSpec((1, E), lambda tb: (0, 0)),
            pl.BlockSpec((1, 1), lambda tb: (0, 0)),
        ],
        out_shape=[
            jax.ShapeDtypeStruct((N_TOK, 1), jnp.int32),
            jax.ShapeDtypeStruct((N_TOK, 1), jnp.int32),
            jax.ShapeDtypeStruct((N_TOK, 1), jnp.int32),
            jax.ShapeDtypeStruct((N_TOK, 1), jnp.int32),
            jax.ShapeDtypeStruct((N_TOK, 1), jnp.float32),
            jax.ShapeDtypeStruct((N_TOK, 1), jnp.float32),
            jax.ShapeDtypeStruct((1, E), jnp.float32),
            jax.ShapeDtypeStruct((1, 1), jnp.float32),
        ],
        scratch_shapes=[
            pltpu.VMEM((1, E), jnp.float32),
            pltpu.VMEM((1, E), jnp.float32),
        ],
    )(x2, gate_w)


def _rowidx_kernel(e1_ref, e2_ref, k1_ref, k2_ref, off_ref, r1_ref, r2_ref):
    lane = jax.lax.broadcasted_iota(jnp.int32, (N_TOK, E), 1)
    off = off_ref[:, :E]
    off1 = jnp.sum(jnp.where(lane == e1_ref[...], off, 0),
                   axis=1, keepdims=True)
    off2 = jnp.sum(jnp.where(lane == e2_ref[...], off, 0),
                   axis=1, keepdims=True)
    r1_ref[...] = off1 + k1_ref[...]
    r2_ref[...] = off2 + k2_ref[...]


def _rowidx(e1, e2, k1, k2, off16):
    return pl.pallas_call(
        _rowidx_kernel,
        out_shape=[
            jax.ShapeDtypeStruct((N_TOK, 1), jnp.int32),
            jax.ShapeDtypeStruct((N_TOK, 1), jnp.int32),
        ],
    )(e1, e2, k1, k2, off16)


def _dispatch_body(x_hbm, r1_hbm, r2_hbm, xd_hbm,
                   xloc, r1v, r2v, sem1, sem2):
    wid = lax.axis_index("s") * 2 + lax.axis_index("c")
    for c in range(TPW // CH):
        base = wid * TPW + c * CH
        pltpu.sync_copy(x_hbm.at[pl.ds(base, CH)], xloc)
        pltpu.sync_copy(r1_hbm.at[pl.ds(base, CH)], r1v)
        pltpu.sync_copy(r2_hbm.at[pl.ds(base, CH)], r2v)
        cp1 = pltpu.async_copy(xloc, xd_hbm.at[r1v], sem1)
        cp2 = pltpu.async_copy(xloc, xd_hbm.at[r2v], sem2)
        cp1.wait()
        cp2.wait()


def _dispatch(x2, r1, r2):
    mesh = plsc.VectorSubcoreMesh(core_axis_name="c", subcore_axis_name="s", num_cores=2, num_subcores=16)
    f = functools.partial(
        pl.kernel, mesh=mesh,
        out_type=jax.ShapeDtypeStruct((NROWS, D), jnp.float32),
        scratch_types=[
            pltpu.VMEM((CH, D), jnp.float32),
            pltpu.VMEM((CH,), jnp.int32),
            pltpu.VMEM((CH,), jnp.int32),
            pltpu.SemaphoreType.DMA,
            pltpu.SemaphoreType.DMA,
        ],
    )(_dispatch_body)
    return f(x2, r1, r2)


def _ffn_kernel(be_ref, nu_ref, x_ref, w1_ref, b1_ref, w2_ref, b2_ref,
                w3_ref, b3_ref, y_ref):
    i = pl.program_id(0)

    @pl.when(i < nu_ref[0])
    def _compute():
        xb = x_ref[...].astype(jnp.bfloat16)
        h = jnp.dot(xb, w1_ref[0].astype(jnp.bfloat16),
                    preferred_element_type=jnp.float32) + b1_ref[0]
        h = _gelu_exact(h)
        h = jnp.dot(h.astype(jnp.bfloat16), w2_ref[0].astype(jnp.bfloat16),
                    preferred_element_type=jnp.float32) + b2_ref[0]
        h = _gelu_exact(h)
        y_ref[...] = jnp.dot(h.astype(jnp.bfloat16),
                             w3_ref[0].astype(jnp.bfloat16),
                             preferred_element_type=jnp.float32) + b3_ref[0]


def _ffn(be, nu, x_disp, w1, b1, w2, b2, w3, b3):
    grid_spec = pltpu.PrefetchScalarGridSpec(
        num_scalar_prefetch=2,
        grid=(NB,),
        in_specs=[
            pl.BlockSpec((TB, D), lambda i, be, nu: (i, 0)),
            pl.BlockSpec((1, D, H), lambda i, be, nu: (be[i], 0, 0)),
            pl.BlockSpec((1, 1, H), lambda i, be, nu: (be[i], 0, 0)),
            pl.BlockSpec((1, H, H), lambda i, be, nu: (be[i], 0, 0)),
            pl.BlockSpec((1, 1, H), lambda i, be, nu: (be[i], 0, 0)),
            pl.BlockSpec((1, H, O), lambda i, be, nu: (be[i], 0, 0)),
            pl.BlockSpec((1, 1, O), lambda i, be, nu: (be[i], 0, 0)),
        ],
        out_specs=pl.BlockSpec((TB, O), lambda i, be, nu: (i, 0)),
    )
    return pl.pallas_call(
        _ffn_kernel,
        grid_spec=grid_spec,
        out_shape=jax.ShapeDtypeStruct((NROWS, O), jnp.float32),
    )(be, nu, x_disp, w1, b1.reshape(E, 1, H), w2, b2.reshape(E, 1, H),
      w3, b3.reshape(E, 1, O))


def _collect_body(yd_hbm, r1_hbm, r2_hbm, ya_hbm, yb_hbm,
                  y1loc, y2loc, r1v, r2v, sem1, sem2):
    wid = lax.axis_index("s") * 2 + lax.axis_index("c")
    for c in range(TPW // CH):
        base = wid * TPW + c * CH
        pltpu.sync_copy(r1_hbm.at[pl.ds(base, CH)], r1v)
        pltpu.sync_copy(r2_hbm.at[pl.ds(base, CH)], r2v)
        cp1 = pltpu.async_copy(yd_hbm.at[r1v], y1loc, sem1)
        cp2 = pltpu.async_copy(yd_hbm.at[r2v], y2loc, sem2)
        cp1.wait()
        pltpu.sync_copy(y1loc, ya_hbm.at[pl.ds(base, CH)])
        cp2.wait()
        pltpu.sync_copy(y2loc, yb_hbm.at[pl.ds(base, CH)])


def _collect(y_disp, r1, r2):
    mesh = plsc.VectorSubcoreMesh(core_axis_name="c", subcore_axis_name="s", num_cores=2, num_subcores=16)
    f = functools.partial(
        pl.kernel, mesh=mesh,
        out_type=[
            jax.ShapeDtypeStruct((N_TOK, O), jnp.float32),
            jax.ShapeDtypeStruct((N_TOK, O), jnp.float32),
        ],
        scratch_types=[
            pltpu.VMEM((CH, O), jnp.float32),
            pltpu.VMEM((CH, O), jnp.float32),
            pltpu.VMEM((CH,), jnp.int32),
            pltpu.VMEM((CH,), jnp.int32),
            pltpu.SemaphoreType.DMA,
            pltpu.SemaphoreType.DMA,
        ],
    )(_collect_body)
    return f(y_disp, r1, r2)


def _combine_kernel(ya_ref, yb_ref, g1_ref, g2_ref, ln_g_ref, ln_b_ref,
                    out_ref):
    a = g1_ref[...] * ya_ref[...] + g2_ref[...] * yb_ref[...]
    mu = jnp.mean(a, axis=1, keepdims=True)
    var = jnp.mean((a - mu) ** 2, axis=1, keepdims=True)
    out_ref[...] = ((a - mu) * jax.lax.rsqrt(var + 1e-5)
                    * ln_g_ref[...] + ln_b_ref[...])


def _combine(ya, yb, g1, g2, ln_g, ln_b):
    return pl.pallas_call(
        _combine_kernel,
        grid=(NTB,),
        in_specs=[
            pl.BlockSpec((TB, O), lambda tb: (tb, 0)),
            pl.BlockSpec((TB, O), lambda tb: (tb, 0)),
            pl.BlockSpec((TB, 1), lambda tb: (tb, 0)),
            pl.BlockSpec((TB, 1), lambda tb: (tb, 0)),
            pl.BlockSpec((1, O), lambda tb: (0, 0)),
            pl.BlockSpec((1, O), lambda tb: (0, 0)),
        ],
        out_specs=pl.BlockSpec((TB, O), lambda tb: (tb, 0)),
        out_shape=jax.ShapeDtypeStruct((N_TOK, O), jnp.float32),
    )(ya, yb, g1, g2, ln_g.reshape(1, O), ln_b.reshape(1, O))


@jax.jit
def kernel(x, gate_w, w1, b1, w2, b2, w3, b3, ln_g, ln_b):
    b, s, d = x.shape
    x2 = x.reshape(b * s, d)

    e1, e2, k1, k2, g1, g2, counts, aux = _router(x2, gate_w)

    c = counts[0].astype(jnp.int32)
    nb_e = (c + TB - 1) // TB
    cumnb = jnp.cumsum(nb_e)
    off16 = jnp.concatenate(
        [(cumnb - nb_e) * TB, jnp.zeros((E,), jnp.int32)])[None, :]
    iota_nb = jnp.arange(NB, dtype=jnp.int32)
    be = jnp.minimum(
        jnp.sum((cumnb[None, :] <= iota_nb[:, None]).astype(jnp.int32),
                axis=1), E - 1).astype(jnp.int32)
    nu = cumnb[E - 1:E]

    r1, r2 = _rowidx(e1, e2, k1, k2, off16)
    r1 = r1.reshape(-1)
    r2 = r2.reshape(-1)
    x_disp = _dispatch(x2, r1, r2)
    y_disp = _ffn(be, nu, x_disp, w1, b1, w2, b2, w3, b3)
    ya, yb = _collect(y_disp, r1, r2)
    out2 = _combine(ya, yb, g1, g2, ln_g, ln_b)
    return out2.reshape(b, s, O), aux[0, 0]


# speedup vs baseline: 4.6723x; 1.0412x over previous
import functools

import jax
import jax.numpy as jnp
from jax import lax
from jax.experimental import pallas as pl
from jax.experimental.pallas import tpu as pltpu
from jax.experimental.pallas import tpu_sc as plsc

E = 8
TOP_K = 2
D = 768
H = 768
O = 768
N_TOK = 4096
TB = 512
NTB = N_TOK // TB
NB = N_TOK * TOP_K // TB + E
NROWS = NB * TB

NW = 32
TPW = N_TOK // NW
CH = 64


def _gelu_exact(h):
    return 0.5 * h * (1.0 + jax.lax.erf(h * 0.7071067811865476))


def _router_kernel(x_ref, gw_ref,
                   g1_ref, g2_ref, r1_ref, r2_ref, counts_ref, aux_ref,
                   cnt_s, use_s, lt_s, e1_s, e2_s, k1_s, k2_s):
    tb = pl.program_id(0)

    @pl.when(tb == 0)
    def _init():
        cnt_s[...] = jnp.zeros((1, E), jnp.float32)
        use_s[...] = jnp.zeros((1, E), jnp.float32)
        row = jax.lax.broadcasted_iota(jnp.int32, (TB, TB), 0)
        col = jax.lax.broadcasted_iota(jnp.int32, (TB, TB), 1)
        lt_s[...] = (col < row).astype(jnp.bfloat16)

    @pl.when(tb < NTB)
    def _block():
        xb = x_ref[...]
        logits = jnp.dot(xb, gw_ref[...], preferred_element_type=jnp.float32)
        lane = jax.lax.broadcasted_iota(jnp.int32, (TB, E), 1)
        v1 = jnp.max(logits, axis=1, keepdims=True)
        i1 = jnp.argmax(logits, axis=1)[:, None]
        masked = jnp.where(lane == i1, -jnp.inf, logits)
        v2 = jnp.max(masked, axis=1, keepdims=True)
        i2 = jnp.argmax(masked, axis=1)[:, None]
        g1 = 1.0 / (1.0 + jnp.exp(v2 - v1))
        g2 = 1.0 - g1

        oh1 = (lane == i1).astype(jnp.float32)
        oh2 = (lane == i2).astype(jnp.float32)
        ohs = oh1 + oh2
        excl = jnp.dot(lt_s[...], ohs.astype(jnp.bfloat16),
                       preferred_element_type=jnp.float32)
        cnt = cnt_s[...]
        rank1 = jnp.sum(oh1 * (cnt + excl), axis=1, keepdims=True)
        rank2 = jnp.sum(oh2 * (cnt + excl + oh1), axis=1, keepdims=True)
        cnt_s[...] = cnt + jnp.sum(ohs, axis=0, keepdims=True)
        use_s[...] = use_s[...] + jnp.sum(oh1 * g1 + oh2 * g2, axis=0,
                                          keepdims=True)

        sl = pl.ds(tb * TB, TB)
        e1_s[sl, :] = i1
        e2_s[sl, :] = i2
        k1_s[sl, :] = rank1.astype(jnp.int32)
        k2_s[sl, :] = rank2.astype(jnp.int32)
        g1_ref[...] = g1
        g2_ref[...] = g2

    @pl.when(tb == NTB)
    def _fin():
        cnt = cnt_s[...]
        counts_ref[...] = cnt
        usage = use_s[...] / N_TOK
        uniform = 1.0 / E
        aux_ref[...] = jnp.sum(uniform * (jnp.log(uniform) - jnp.log(usage)),
                               axis=1, keepdims=True)
        nb_e = jnp.floor((cnt + (TB - 1)) * (1.0 / TB))
        r8 = jax.lax.broadcasted_iota(jnp.int32, (E, E), 0)
        c8 = jax.lax.broadcasted_iota(jnp.int32, (E, E), 1)
        le = (r8 <= c8).astype(jnp.float32)
        cum = jnp.dot(nb_e, le, preferred_element_type=jnp.float32)
        off = ((cum - nb_e) * TB).astype(jnp.int32)
        lane = jax.lax.broadcasted_iota(jnp.int32, (N_TOK, E), 1)
        off1 = jnp.sum(jnp.where(lane == e1_s[...], off, 0),
                       axis=1, keepdims=True)
        off2 = jnp.sum(jnp.where(lane == e2_s[...], off, 0),
                       axis=1, keepdims=True)
        r1_ref[...] = off1 + k1_s[...]
        r2_ref[...] = off2 + k2_s[...]


def _router(x2, gate_w):
    return pl.pallas_call(
        _router_kernel,
        grid=(NTB + 1,),
        in_specs=[
            pl.BlockSpec((TB, D), lambda tb: (jnp.minimum(tb, NTB - 1), 0)),
            pl.BlockSpec((D, E), lambda tb: (0, 0)),
        ],
        out_specs=[
            pl.BlockSpec((TB, 1), lambda tb: (jnp.minimum(tb, NTB - 1), 0)),
            pl.BlockSpec((TB, 1), lambda tb: (jnp.minimum(tb, NTB - 1), 0)),
            pl.BlockSpec((N_TOK, 1), lambda tb: (0, 0)),
            pl.BlockSpec((N_TOK, 1), lambda tb: (0, 0)),
            pl.BlockSpec((1, E), lambda tb: (0, 0)),
            pl.BlockSpec((1, 1), lambda tb: (0, 0)),
        ],
        out_shape=[
            jax.ShapeDtypeStruct((N_TOK, 1), jnp.float32),
            jax.ShapeDtypeStruct((N_TOK, 1), jnp.float32),
            jax.ShapeDtypeStruct((N_TOK, 1), jnp.int32),
            jax.ShapeDtypeStruct((N_TOK, 1), jnp.int32),
            jax.ShapeDtypeStruct((1, E), jnp.float32),
            jax.ShapeDtypeStruct((1, 1), jnp.float32),
        ],
        scratch_shapes=[
            pltpu.VMEM((1, E), jnp.float32),
            pltpu.VMEM((1, E), jnp.float32),
            pltpu.VMEM((TB, TB), jnp.bfloat16),
            pltpu.VMEM((N_TOK, 1), jnp.int32),
            pltpu.VMEM((N_TOK, 1), jnp.int32),
            pltpu.VMEM((N_TOK, 1), jnp.int32),
            pltpu.VMEM((N_TOK, 1), jnp.int32),
        ],
    )(x2, gate_w)


def _dispatch_body(x_hbm, r1_hbm, r2_hbm, xd_hbm,
                   xloc, r1v, r2v, sem1, sem2):
    wid = lax.axis_index("s") * 2 + lax.axis_index("c")
    for c in range(TPW // CH):
        base = wid * TPW + c * CH
        pltpu.sync_copy(x_hbm.at[pl.ds(base, CH)], xloc)
        pltpu.sync_copy(r1_hbm.at[pl.ds(base, CH)], r1v)
        pltpu.sync_copy(r2_hbm.at[pl.ds(base, CH)], r2v)
        cp1 = pltpu.async_copy(xloc, xd_hbm.at[r1v], sem1)
        cp2 = pltpu.async_copy(xloc, xd_hbm.at[r2v], sem2)
        cp1.wait()
        cp2.wait()


def _dispatch(x2, r1, r2):
    mesh = plsc.VectorSubcoreMesh(core_axis_name="c", subcore_axis_name="s", num_cores=2, num_subcores=16)
    f = functools.partial(
        pl.kernel, mesh=mesh,
        out_type=jax.ShapeDtypeStruct((NROWS, D), jnp.float32),
        scratch_types=[
            pltpu.VMEM((CH, D), jnp.float32),
            pltpu.VMEM((CH,), jnp.int32),
            pltpu.VMEM((CH,), jnp.int32),
            pltpu.SemaphoreType.DMA,
            pltpu.SemaphoreType.DMA,
        ],
    )(_dispatch_body)
    return f(x2, r1, r2)


def _ffn_kernel(be_ref, nu_ref, x_ref, w1_ref, b1_ref, w2_ref, b2_ref,
                w3_ref, b3_ref, y_ref, w1b, w2b, w3b):
    i = pl.program_id(0)
    nu = nu_ref[0]
    new_expert = jnp.logical_or(
        i == 0, be_ref[i] != be_ref[jnp.maximum(i - 1, 0)])

    @pl.when(jnp.logical_and(i < nu, new_expert))
    def _cast_weights():
        w1b[...] = w1_ref[0].astype(jnp.bfloat16)
        w2b[...] = w2_ref[0].astype(jnp.bfloat16)
        w3b[...] = w3_ref[0].astype(jnp.bfloat16)

    @pl.when(i < nu)
    def _compute():
        xb = x_ref[...].astype(jnp.bfloat16)
        h = jnp.dot(xb, w1b[...],
                    preferred_element_type=jnp.float32) + b1_ref[0]
        h = _gelu_exact(h)
        h = jnp.dot(h.astype(jnp.bfloat16), w2b[...],
                    preferred_element_type=jnp.float32) + b2_ref[0]
        h = _gelu_exact(h)
        y_ref[...] = jnp.dot(h.astype(jnp.bfloat16), w3b[...],
                             preferred_element_type=jnp.float32) + b3_ref[0]


def _ffn(be, nu, x_disp, w1, b1, w2, b2, w3, b3):
    grid_spec = pltpu.PrefetchScalarGridSpec(
        num_scalar_prefetch=2,
        grid=(NB,),
        in_specs=[
            pl.BlockSpec((TB, D),
                         lambda i, be, nu: (jnp.minimum(i, nu[0] - 1), 0)),
            pl.BlockSpec((1, D, H), lambda i, be, nu: (be[i], 0, 0)),
            pl.BlockSpec((1, 1, H), lambda i, be, nu: (be[i], 0, 0)),
            pl.BlockSpec((1, H, H), lambda i, be, nu: (be[i], 0, 0)),
            pl.BlockSpec((1, 1, H), lambda i, be, nu: (be[i], 0, 0)),
            pl.BlockSpec((1, H, O), lambda i, be, nu: (be[i], 0, 0)),
            pl.BlockSpec((1, 1, O), lambda i, be, nu: (be[i], 0, 0)),
        ],
        out_specs=pl.BlockSpec((TB, O), lambda i, be, nu: (i, 0)),
        scratch_shapes=[
            pltpu.VMEM((D, H), jnp.bfloat16),
            pltpu.VMEM((H, H), jnp.bfloat16),
            pltpu.VMEM((H, O), jnp.bfloat16),
        ],
    )
    return pl.pallas_call(
        _ffn_kernel,
        grid_spec=grid_spec,
        out_shape=jax.ShapeDtypeStruct((NROWS, O), jnp.float32),
    )(be, nu, x_disp, w1, b1.reshape(E, 1, H), w2, b2.reshape(E, 1, H),
      w3, b3.reshape(E, 1, O))


def _collect_body(yd_hbm, r1_hbm, r2_hbm, ya_hbm, yb_hbm,
                  y1loc, y2loc, r1v, r2v, sem1, sem2):
    wid = lax.axis_index("s") * 2 + lax.axis_index("c")
    for c in range(TPW // CH):
        base = wid * TPW + c * CH
        pltpu.sync_copy(r1_hbm.at[pl.ds(base, CH)], r1v)
        pltpu.sync_copy(r2_hbm.at[pl.ds(base, CH)], r2v)
        cp1 = pltpu.async_copy(yd_hbm.at[r1v], y1loc, sem1)
        cp2 = pltpu.async_copy(yd_hbm.at[r2v], y2loc, sem2)
        cp1.wait()
        pltpu.sync_copy(y1loc, ya_hbm.at[pl.ds(base, CH)])
        cp2.wait()
        pltpu.sync_copy(y2loc, yb_hbm.at[pl.ds(base, CH)])


def _collect(y_disp, r1, r2):
    mesh = plsc.VectorSubcoreMesh(core_axis_name="c", subcore_axis_name="s", num_cores=2, num_subcores=16)
    f = functools.partial(
        pl.kernel, mesh=mesh,
        out_type=[
            jax.ShapeDtypeStruct((N_TOK, O), jnp.float32),
            jax.ShapeDtypeStruct((N_TOK, O), jnp.float32),
        ],
        scratch_types=[
            pltpu.VMEM((CH, O), jnp.float32),
            pltpu.VMEM((CH, O), jnp.float32),
            pltpu.VMEM((CH,), jnp.int32),
            pltpu.VMEM((CH,), jnp.int32),
            pltpu.SemaphoreType.DMA,
            pltpu.SemaphoreType.DMA,
        ],
    )(_collect_body)
    return f(y_disp, r1, r2)


def _combine_kernel(ya_ref, yb_ref, g1_ref, g2_ref, ln_g_ref, ln_b_ref,
                    out_ref):
    a = g1_ref[...] * ya_ref[...] + g2_ref[...] * yb_ref[...]
    mu = jnp.mean(a, axis=1, keepdims=True)
    var = jnp.mean((a - mu) ** 2, axis=1, keepdims=True)
    out_ref[...] = ((a - mu) * jax.lax.rsqrt(var + 1e-5)
                    * ln_g_ref[...] + ln_b_ref[...])


def _combine(ya, yb, g1, g2, ln_g, ln_b):
    return pl.pallas_call(
        _combine_kernel,
        grid=(NTB,),
        in_specs=[
            pl.BlockSpec((TB, O), lambda tb: (tb, 0)),
            pl.BlockSpec((TB, O), lambda tb: (tb, 0)),
            pl.BlockSpec((TB, 1), lambda tb: (tb, 0)),
            pl.BlockSpec((TB, 1), lambda tb: (tb, 0)),
            pl.BlockSpec((1, O), lambda tb: (0, 0)),
            pl.BlockSpec((1, O), lambda tb: (0, 0)),
        ],
        out_specs=pl.BlockSpec((TB, O), lambda tb: (tb, 0)),
        out_shape=jax.ShapeDtypeStruct((N_TOK, O), jnp.float32),
    )(ya, yb, g1, g2, ln_g.reshape(1, O), ln_b.reshape(1, O))


@jax.jit
def kernel(x, gate_w, w1, b1, w2, b2, w3, b3, ln_g, ln_b):
    b, s, d = x.shape
    x2 = x.reshape(b * s, d)

    g1, g2, r1, r2, counts, aux = _router(x2, gate_w)

    c = counts[0].astype(jnp.int32)
    nb_e = (c + TB - 1) // TB
    cumnb = jnp.cumsum(nb_e)
    iota_nb = jnp.arange(NB, dtype=jnp.int32)
    be = jnp.minimum(
        jnp.sum((cumnb[None, :] <= iota_nb[:, None]).astype(jnp.int32),
                axis=1), E - 1).astype(jnp.int32)
    nu = cumnb[E - 1:E]

    r1 = r1.reshape(-1)
    r2 = r2.reshape(-1)
    x_disp = _dispatch(x2, r1, r2)
    y_disp = _ffn(be, nu, x_disp, w1, b1, w2, b2, w3, b3)
    ya, yb = _collect(y_disp, r1, r2)
    out2 = _combine(ya, yb, g1, g2, ln_g, ln_b)
    return out2.reshape(b, s, O), aux[0, 0]


# speedup vs baseline: 5.3265x; 1.1400x over previous
import functools

import jax
import jax.numpy as jnp
from jax import lax
from jax.experimental import pallas as pl
from jax.experimental.pallas import tpu as pltpu
from jax.experimental.pallas import tpu_sc as plsc

E = 8
TOP_K = 2
D = 768
H = 768
O = 768
N_TOK = 4096
TB = 512
NTB = N_TOK // TB
NB = N_TOK * TOP_K // TB + E
NROWS = NB * TB

NW = 32
TPW = N_TOK // NW
CH = 64


def _gelu_exact(h):
    return 0.5 * h * (1.0 + jax.lax.erf(h * 0.7071067811865476))


D2 = D // 2


def _pack_bf16(a, b):
    au = jax.lax.bitcast_convert_type(a, jnp.uint32) + 0x8000
    bu = jax.lax.bitcast_convert_type(b, jnp.uint32) + 0x8000
    pu = (au & jnp.uint32(0xFFFF0000)) | (bu >> 16)
    return jax.lax.bitcast_convert_type(pu, jnp.int32)


def _unpack_f32(p):
    pu = jax.lax.bitcast_convert_type(p, jnp.uint32)
    a = jax.lax.bitcast_convert_type(pu & jnp.uint32(0xFFFF0000), jnp.float32)
    b = jax.lax.bitcast_convert_type(pu << 16, jnp.float32)
    return a, b


def _router_kernel(x_ref, gw_ref,
                   xp_ref, g1_ref, g2_ref, r1_ref, r2_ref, counts_ref,
                   aux_ref,
                   cnt_s, use_s, lt_s, e1_s, e2_s, k1_s, k2_s):
    tb = pl.program_id(0)

    @pl.when(tb == 0)
    def _init():
        cnt_s[...] = jnp.zeros((1, E), jnp.float32)
        use_s[...] = jnp.zeros((1, E), jnp.float32)
        row = jax.lax.broadcasted_iota(jnp.int32, (TB, TB), 0)
        col = jax.lax.broadcasted_iota(jnp.int32, (TB, TB), 1)
        lt_s[...] = (col < row).astype(jnp.bfloat16)

    @pl.when(tb < NTB)
    def _block():
        xb = x_ref[...]
        xp_ref[...] = _pack_bf16(xb[:, :D2], xb[:, D2:])
        logits = jnp.dot(xb, gw_ref[...], preferred_element_type=jnp.float32)
        lane = jax.lax.broadcasted_iota(jnp.int32, (TB, E), 1)
        v1 = jnp.max(logits, axis=1, keepdims=True)
        i1 = jnp.argmax(logits, axis=1)[:, None]
        masked = jnp.where(lane == i1, -jnp.inf, logits)
        v2 = jnp.max(masked, axis=1, keepdims=True)
        i2 = jnp.argmax(masked, axis=1)[:, None]
        g1 = 1.0 / (1.0 + jnp.exp(v2 - v1))
        g2 = 1.0 - g1

        oh1 = (lane == i1).astype(jnp.float32)
        oh2 = (lane == i2).astype(jnp.float32)
        ohs = oh1 + oh2
        excl = jnp.dot(lt_s[...], ohs.astype(jnp.bfloat16),
                       preferred_element_type=jnp.float32)
        cnt = cnt_s[...]
        rank1 = jnp.sum(oh1 * (cnt + excl), axis=1, keepdims=True)
        rank2 = jnp.sum(oh2 * (cnt + excl + oh1), axis=1, keepdims=True)
        cnt_s[...] = cnt + jnp.sum(ohs, axis=0, keepdims=True)
        use_s[...] = use_s[...] + jnp.sum(oh1 * g1 + oh2 * g2, axis=0,
                                          keepdims=True)

        sl = pl.ds(tb * TB, TB)
        e1_s[sl, :] = i1
        e2_s[sl, :] = i2
        k1_s[sl, :] = rank1.astype(jnp.int32)
        k2_s[sl, :] = rank2.astype(jnp.int32)
        g1_ref[...] = g1
        g2_ref[...] = g2

    @pl.when(tb == NTB)
    def _fin():
        cnt = cnt_s[...]
        counts_ref[...] = cnt
        usage = use_s[...] / N_TOK
        uniform = 1.0 / E
        aux_ref[...] = jnp.sum(uniform * (jnp.log(uniform) - jnp.log(usage)),
                               axis=1, keepdims=True)
        nb_e = jnp.floor((cnt + (TB - 1)) * (1.0 / TB))
        r8 = jax.lax.broadcasted_iota(jnp.int32, (E, E), 0)
        c8 = jax.lax.broadcasted_iota(jnp.int32, (E, E), 1)
        le = (r8 <= c8).astype(jnp.float32)
        cum = jnp.dot(nb_e, le, preferred_element_type=jnp.float32)
        off = ((cum - nb_e) * TB).astype(jnp.int32)
        lane = jax.lax.broadcasted_iota(jnp.int32, (N_TOK, E), 1)
        off1 = jnp.sum(jnp.where(lane == e1_s[...], off, 0),
                       axis=1, keepdims=True)
        off2 = jnp.sum(jnp.where(lane == e2_s[...], off, 0),
                       axis=1, keepdims=True)
        r1_ref[...] = off1 + k1_s[...]
        r2_ref[...] = off2 + k2_s[...]


def _router(x2, gate_w):
    return pl.pallas_call(
        _router_kernel,
        grid=(NTB + 1,),
        in_specs=[
            pl.BlockSpec((TB, D), lambda tb: (jnp.minimum(tb, NTB - 1), 0)),
            pl.BlockSpec((D, E), lambda tb: (0, 0)),
        ],
        out_specs=[
            pl.BlockSpec((TB, D2), lambda tb: (jnp.minimum(tb, NTB - 1), 0)),
            pl.BlockSpec((TB, 1), lambda tb: (jnp.minimum(tb, NTB - 1), 0)),
            pl.BlockSpec((TB, 1), lambda tb: (jnp.minimum(tb, NTB - 1), 0)),
            pl.BlockSpec((N_TOK, 1), lambda tb: (0, 0)),
            pl.BlockSpec((N_TOK, 1), lambda tb: (0, 0)),
            pl.BlockSpec((1, E), lambda tb: (0, 0)),
            pl.BlockSpec((1, 1), lambda tb: (0, 0)),
        ],
        out_shape=[
            jax.ShapeDtypeStruct((N_TOK, D2), jnp.int32),
            jax.ShapeDtypeStruct((N_TOK, 1), jnp.float32),
            jax.ShapeDtypeStruct((N_TOK, 1), jnp.float32),
            jax.ShapeDtypeStruct((N_TOK, 1), jnp.int32),
            jax.ShapeDtypeStruct((N_TOK, 1), jnp.int32),
            jax.ShapeDtypeStruct((1, E), jnp.float32),
            jax.ShapeDtypeStruct((1, 1), jnp.float32),
        ],
        scratch_shapes=[
            pltpu.VMEM((1, E), jnp.float32),
            pltpu.VMEM((1, E), jnp.float32),
            pltpu.VMEM((TB, TB), jnp.bfloat16),
            pltpu.VMEM((N_TOK, 1), jnp.int32),
            pltpu.VMEM((N_TOK, 1), jnp.int32),
            pltpu.VMEM((N_TOK, 1), jnp.int32),
            pltpu.VMEM((N_TOK, 1), jnp.int32),
        ],
    )(x2, gate_w)


def _dispatch_body(x_hbm, r1_hbm, r2_hbm, xd_hbm,
                   xloc, r1v, r2v, sem1, sem2):
    wid = lax.axis_index("s") * 2 + lax.axis_index("c")
    for c in range(TPW // CH):
        base = wid * TPW + c * CH
        pltpu.sync_copy(x_hbm.at[pl.ds(base, CH)], xloc)
        pltpu.sync_copy(r1_hbm.at[pl.ds(base, CH)], r1v)
        pltpu.sync_copy(r2_hbm.at[pl.ds(base, CH)], r2v)
        cp1 = pltpu.async_copy(xloc, xd_hbm.at[r1v], sem1)
        cp2 = pltpu.async_copy(xloc, xd_hbm.at[r2v], sem2)
        cp1.wait()
        cp2.wait()


def _dispatch(xp, r1, r2):
    mesh = plsc.VectorSubcoreMesh(core_axis_name="c", subcore_axis_name="s", num_cores=2, num_subcores=16)
    f = functools.partial(
        pl.kernel, mesh=mesh,
        out_type=jax.ShapeDtypeStruct((NROWS, D2), jnp.int32),
        scratch_types=[
            pltpu.VMEM((CH, D2), jnp.int32),
            pltpu.VMEM((CH,), jnp.int32),
            pltpu.VMEM((CH,), jnp.int32),
            pltpu.SemaphoreType.DMA,
            pltpu.SemaphoreType.DMA,
        ],
    )(_dispatch_body)
    return f(xp, r1, r2)


def _ffn_kernel(be_ref, nu_ref, x_ref, w1_ref, b1_ref, w2_ref, b2_ref,
                w3_ref, b3_ref, y_ref, w1b, w2b, w3b):
    i = pl.program_id(0)
    nu = nu_ref[0]
    new_expert = jnp.logical_or(
        i == 0, be_ref[i] != be_ref[jnp.maximum(i - 1, 0)])

    @pl.when(jnp.logical_and(i < nu, new_expert))
    def _cast_weights():
        w1b[...] = w1_ref[0].astype(jnp.bfloat16)
        w2b[...] = w2_ref[0].astype(jnp.bfloat16)
        w3b[...] = w3_ref[0].astype(jnp.bfloat16)

    @pl.when(i < nu)
    def _compute():
        xa, xc = _unpack_f32(x_ref[...])
        xb = jnp.concatenate([xa, xc], axis=1).astype(jnp.bfloat16)
        h = jnp.dot(xb, w1b[...],
                    preferred_element_type=jnp.float32) + b1_ref[0]
        h = _gelu_exact(h)
        h = jnp.dot(h.astype(jnp.bfloat16), w2b[...],
                    preferred_element_type=jnp.float32) + b2_ref[0]
        h = _gelu_exact(h)
        y = jnp.dot(h.astype(jnp.bfloat16), w3b[...],
                    preferred_element_type=jnp.float32) + b3_ref[0]
        y_ref[...] = _pack_bf16(y[:, :D2], y[:, D2:])


def _ffn(be, nu, x_disp, w1, b1, w2, b2, w3, b3):
    grid_spec = pltpu.PrefetchScalarGridSpec(
        num_scalar_prefetch=2,
        grid=(NB,),
        in_specs=[
            pl.BlockSpec((TB, D2),
                         lambda i, be, nu: (jnp.minimum(i, nu[0] - 1), 0)),
            pl.BlockSpec((1, D, H), lambda i, be, nu: (be[i], 0, 0)),
            pl.BlockSpec((1, 1, H), lambda i, be, nu: (be[i], 0, 0)),
            pl.BlockSpec((1, H, H), lambda i, be, nu: (be[i], 0, 0)),
            pl.BlockSpec((1, 1, H), lambda i, be, nu: (be[i], 0, 0)),
            pl.BlockSpec((1, H, O), lambda i, be, nu: (be[i], 0, 0)),
            pl.BlockSpec((1, 1, O), lambda i, be, nu: (be[i], 0, 0)),
        ],
        out_specs=pl.BlockSpec((TB, D2), lambda i, be, nu: (i, 0)),
        scratch_shapes=[
            pltpu.VMEM((D, H), jnp.bfloat16),
            pltpu.VMEM((H, H), jnp.bfloat16),
            pltpu.VMEM((H, O), jnp.bfloat16),
        ],
    )
    return pl.pallas_call(
        _ffn_kernel,
        grid_spec=grid_spec,
        out_shape=jax.ShapeDtypeStruct((NROWS, D2), jnp.int32),
    )(be, nu, x_disp, w1, b1.reshape(E, 1, H), w2, b2.reshape(E, 1, H),
      w3, b3.reshape(E, 1, O))


def _collect_body(yd_hbm, r1_hbm, r2_hbm, ya_hbm, yb_hbm,
                  y1loc, y2loc, r1v, r2v, sem1, sem2):
    wid = lax.axis_index("s") * 2 + lax.axis_index("c")
    for c in range(TPW // CH):
        base = wid * TPW + c * CH
        pltpu.sync_copy(r1_hbm.at[pl.ds(base, CH)], r1v)
        pltpu.sync_copy(r2_hbm.at[pl.ds(base, CH)], r2v)
        cp1 = pltpu.async_copy(yd_hbm.at[r1v], y1loc, sem1)
        cp2 = pltpu.async_copy(yd_hbm.at[r2v], y2loc, sem2)
        cp1.wait()
        pltpu.sync_copy(y1loc, ya_hbm.at[pl.ds(base, CH)])
        cp2.wait()
        pltpu.sync_copy(y2loc, yb_hbm.at[pl.ds(base, CH)])


def _collect(y_disp, r1, r2):
    mesh = plsc.VectorSubcoreMesh(core_axis_name="c", subcore_axis_name="s", num_cores=2, num_subcores=16)
    f = functools.partial(
        pl.kernel, mesh=mesh,
        out_type=[
            jax.ShapeDtypeStruct((N_TOK, D2), jnp.int32),
            jax.ShapeDtypeStruct((N_TOK, D2), jnp.int32),
        ],
        scratch_types=[
            pltpu.VMEM((CH, D2), jnp.int32),
            pltpu.VMEM((CH, D2), jnp.int32),
            pltpu.VMEM((CH,), jnp.int32),
            pltpu.VMEM((CH,), jnp.int32),
            pltpu.SemaphoreType.DMA,
            pltpu.SemaphoreType.DMA,
        ],
    )(_collect_body)
    return f(y_disp, r1, r2)


def _combine_kernel(ya_ref, yb_ref, g1_ref, g2_ref, ln_g_ref, ln_b_ref,
                    out_ref):
    ya1, ya2 = _unpack_f32(ya_ref[...])
    yaf = jnp.concatenate([ya1, ya2], axis=1)
    yb1, yb2 = _unpack_f32(yb_ref[...])
    ybf = jnp.concatenate([yb1, yb2], axis=1)
    a = g1_ref[...] * yaf + g2_ref[...] * ybf
    mu = jnp.mean(a, axis=1, keepdims=True)
    var = jnp.mean((a - mu) ** 2, axis=1, keepdims=True)
    out_ref[...] = ((a - mu) * jax.lax.rsqrt(var + 1e-5)
                    * ln_g_ref[...] + ln_b_ref[...])


def _combine(ya, yb, g1, g2, ln_g, ln_b):
    return pl.pallas_call(
        _combine_kernel,
        grid=(NTB,),
        in_specs=[
            pl.BlockSpec((TB, D2), lambda tb: (tb, 0)),
            pl.BlockSpec((TB, D2), lambda tb: (tb, 0)),
            pl.BlockSpec((TB, 1), lambda tb: (tb, 0)),
            pl.BlockSpec((TB, 1), lambda tb: (tb, 0)),
            pl.BlockSpec((1, O), lambda tb: (0, 0)),
            pl.BlockSpec((1, O), lambda tb: (0, 0)),
        ],
        out_specs=pl.BlockSpec((TB, O), lambda tb: (tb, 0)),
        out_shape=jax.ShapeDtypeStruct((N_TOK, O), jnp.float32),
    )(ya, yb, g1, g2, ln_g.reshape(1, O), ln_b.reshape(1, O))


@jax.jit
def kernel(x, gate_w, w1, b1, w2, b2, w3, b3, ln_g, ln_b):
    b, s, d = x.shape
    x2 = x.reshape(b * s, d)

    xp, g1, g2, r1, r2, counts, aux = _router(x2, gate_w)

    c = counts[0].astype(jnp.int32)
    nb_e = (c + TB - 1) // TB
    cumnb = jnp.cumsum(nb_e)
    iota_nb = jnp.arange(NB, dtype=jnp.int32)
    be = jnp.minimum(
        jnp.sum((cumnb[None, :] <= iota_nb[:, None]).astype(jnp.int32),
                axis=1), E - 1).astype(jnp.int32)
    nu = cumnb[E - 1:E]

    r1 = r1.reshape(-1)
    r2 = r2.reshape(-1)
    x_disp = _dispatch(xp, r1, r2)
    y_disp = _ffn(be, nu, x_disp, w1, b1, w2, b2, w3, b3)
    ya, yb = _collect(y_disp, r1, r2)
    out2 = _combine(ya, yb, g1, g2, ln_g, ln_b)
    return out2.reshape(b, s, O), aux[0, 0]


# speedup vs baseline: 5.4673x; 1.0264x over previous
import functools

import jax
import jax.numpy as jnp
from jax import lax
from jax.experimental import pallas as pl
from jax.experimental.pallas import tpu as pltpu
from jax.experimental.pallas import tpu_sc as plsc

E = 8
TOP_K = 2
D = 768
H = 768
O = 768
N_TOK = 4096
TB = 512
NTB = N_TOK // TB
NB = N_TOK * TOP_K // TB + E
CAPB = N_TOK // TB
NROWS = (E * CAPB + 1) * TB

NW = 32
TPW = N_TOK // NW
CH = 64


def _gelu_exact(h):
    return 0.5 * h * (1.0 + jax.lax.erf(h * 0.7071067811865476))


D2 = D // 2


def _pack_bf16(a, b):
    au = jax.lax.bitcast_convert_type(a, jnp.uint32) + 0x8000
    bu = jax.lax.bitcast_convert_type(b, jnp.uint32) + 0x8000
    pu = (au & jnp.uint32(0xFFFF0000)) | (bu >> 16)
    return jax.lax.bitcast_convert_type(pu, jnp.int32)


def _unpack_f32(p):
    pu = jax.lax.bitcast_convert_type(p, jnp.uint32)
    a = jax.lax.bitcast_convert_type(pu & jnp.uint32(0xFFFF0000), jnp.float32)
    b = jax.lax.bitcast_convert_type(pu << 16, jnp.float32)
    return a, b


def _router_kernel(x_ref, gw_ref,
                   xp_ref, g1_ref, g2_ref, r1_ref, r2_ref, rowmap_ref,
                   wmap_ref, nu_ref, aux_ref,
                   cnt_s, use_s, lt_s):
    tb = pl.program_id(0)

    @pl.when(tb == 0)
    def _init():
        cnt_s[...] = jnp.zeros((1, E), jnp.float32)
        use_s[...] = jnp.zeros((1, E), jnp.float32)
        row = jax.lax.broadcasted_iota(jnp.int32, (TB, TB), 0)
        col = jax.lax.broadcasted_iota(jnp.int32, (TB, TB), 1)
        lt_s[...] = (col < row).astype(jnp.bfloat16)

    @pl.when(tb < NTB)
    def _block():
        xb = x_ref[...]
        xp_ref[...] = _pack_bf16(xb[:, :D2], xb[:, D2:])
        logits = jnp.dot(xb, gw_ref[...], preferred_element_type=jnp.float32)
        lane = jax.lax.broadcasted_iota(jnp.int32, (TB, E), 1)
        v1 = jnp.max(logits, axis=1, keepdims=True)
        i1 = jnp.argmax(logits, axis=1)[:, None]
        masked = jnp.where(lane == i1, -jnp.inf, logits)
        v2 = jnp.max(masked, axis=1, keepdims=True)
        i2 = jnp.argmax(masked, axis=1)[:, None]
        g1 = 1.0 / (1.0 + jnp.exp(v2 - v1))
        g2 = 1.0 - g1

        oh1 = (lane == i1).astype(jnp.float32)
        oh2 = (lane == i2).astype(jnp.float32)
        ohs = oh1 + oh2
        excl = jnp.dot(lt_s[...], ohs.astype(jnp.bfloat16),
                       preferred_element_type=jnp.float32)
        cnt = cnt_s[...]
        rank1 = jnp.sum(oh1 * (cnt + excl), axis=1, keepdims=True)
        rank2 = jnp.sum(oh2 * (cnt + excl + oh1), axis=1, keepdims=True)
        cnt_s[...] = cnt + jnp.sum(ohs, axis=0, keepdims=True)
        use_s[...] = use_s[...] + jnp.sum(oh1 * g1 + oh2 * g2, axis=0,
                                          keepdims=True)

        r1_ref[...] = i1 * N_TOK + rank1.astype(jnp.int32)
        r2_ref[...] = i2 * N_TOK + rank2.astype(jnp.int32)
        g1_ref[...] = g1
        g2_ref[...] = g2

    @pl.when(tb == NTB)
    def _fin():
        cnt = cnt_s[...]
        usage = use_s[...] / N_TOK
        uniform = 1.0 / E
        aux_ref[...] = jnp.sum(uniform * (jnp.log(uniform) - jnp.log(usage)),
                               axis=1, keepdims=True)
        nb_e = jnp.floor((cnt + (TB - 1)) * (1.0 / TB))
        r8 = jax.lax.broadcasted_iota(jnp.int32, (E, E), 0)
        c8 = jax.lax.broadcasted_iota(jnp.int32, (E, E), 1)
        le = (r8 <= c8).astype(jnp.float32)
        cum = jnp.dot(nb_e, le, preferred_element_type=jnp.float32)
        iota_nb = jax.lax.broadcasted_iota(
            jnp.int32, (1, NB), 1).astype(jnp.float32)
        e_sel = jnp.zeros((1, NB), jnp.float32)
        cexcl_sel = jnp.zeros((1, NB), jnp.float32)
        for e in range(E):
            cum_e = cum[0:1, e:e + 1]
            e_sel = e_sel + (iota_nb >= cum_e).astype(jnp.float32)
        for e in range(E):
            cexcl_e = cum[0:1, e:e + 1] - nb_e[0:1, e:e + 1]
            e_eq = (e_sel == e).astype(jnp.float32)
            cexcl_sel = cexcl_sel + e_eq * cexcl_e
        nu = cum[0:1, E - 1:E]
        live = iota_nb < nu
        boff = iota_nb - cexcl_sel
        rowmap = jnp.where(live, e_sel * CAPB + boff,
                           float(E * CAPB)).astype(jnp.int32)
        wmap = jnp.minimum(e_sel, E - 1).astype(jnp.int32)
        rowmap_ref[...] = rowmap
        wmap_ref[...] = wmap
        nu_ref[...] = nu.astype(jnp.int32)


def _router(x2, gate_w):
    return pl.pallas_call(
        _router_kernel,
        grid=(NTB + 1,),
        in_specs=[
            pl.BlockSpec((TB, D), lambda tb: (jnp.minimum(tb, NTB - 1), 0)),
            pl.BlockSpec((D, E), lambda tb: (0, 0)),
        ],
        out_specs=[
            pl.BlockSpec((TB, D2), lambda tb: (jnp.minimum(tb, NTB - 1), 0)),
            pl.BlockSpec((TB, 1), lambda tb: (jnp.minimum(tb, NTB - 1), 0)),
            pl.BlockSpec((TB, 1), lambda tb: (jnp.minimum(tb, NTB - 1), 0)),
            pl.BlockSpec((TB, 1), lambda tb: (jnp.minimum(tb, NTB - 1), 0)),
            pl.BlockSpec((TB, 1), lambda tb: (jnp.minimum(tb, NTB - 1), 0)),
            pl.BlockSpec((1, NB), lambda tb: (0, 0)),
            pl.BlockSpec((1, NB), lambda tb: (0, 0)),
            pl.BlockSpec((1, 1), lambda tb: (0, 0)),
            pl.BlockSpec((1, 1), lambda tb: (0, 0)),
        ],
        out_shape=[
            jax.ShapeDtypeStruct((N_TOK, D2), jnp.int32),
            jax.ShapeDtypeStruct((N_TOK, 1), jnp.float32),
            jax.ShapeDtypeStruct((N_TOK, 1), jnp.float32),
            jax.ShapeDtypeStruct((N_TOK, 1), jnp.int32),
            jax.ShapeDtypeStruct((N_TOK, 1), jnp.int32),
            jax.ShapeDtypeStruct((1, NB), jnp.int32),
            jax.ShapeDtypeStruct((1, NB), jnp.int32),
            jax.ShapeDtypeStruct((1, 1), jnp.int32),
            jax.ShapeDtypeStruct((1, 1), jnp.float32),
        ],
        scratch_shapes=[
            pltpu.VMEM((1, E), jnp.float32),
            pltpu.VMEM((1, E), jnp.float32),
            pltpu.VMEM((TB, TB), jnp.bfloat16),
        ],
    )(x2, gate_w)


def _dispatch_body(x_hbm, r1_hbm, r2_hbm, xd_hbm,
                   xloc, r1v, r2v, sem1, sem2):
    wid = lax.axis_index("s") * 2 + lax.axis_index("c")
    for c in range(TPW // CH):
        base = wid * TPW + c * CH
        pltpu.sync_copy(x_hbm.at[pl.ds(base, CH)], xloc)
        pltpu.sync_copy(r1_hbm.at[pl.ds(base, CH)], r1v)
        pltpu.sync_copy(r2_hbm.at[pl.ds(base, CH)], r2v)
        cp1 = pltpu.async_copy(xloc, xd_hbm.at[r1v], sem1)
        cp2 = pltpu.async_copy(xloc, xd_hbm.at[r2v], sem2)
        cp1.wait()
        cp2.wait()


def _dispatch(xp, r1, r2):
    mesh = plsc.VectorSubcoreMesh(core_axis_name="c", subcore_axis_name="s", num_cores=2, num_subcores=16)
    f = functools.partial(
        pl.kernel, mesh=mesh,
        out_type=jax.ShapeDtypeStruct((NROWS, D2), jnp.int32),
        scratch_types=[
            pltpu.VMEM((CH, D2), jnp.int32),
            pltpu.VMEM((CH,), jnp.int32),
            pltpu.VMEM((CH,), jnp.int32),
            pltpu.SemaphoreType.DMA,
            pltpu.SemaphoreType.DMA,
        ],
    )(_dispatch_body)
    return f(xp, r1, r2)


def _ffn_kernel(rm_ref, wm_ref, nu_ref, x_ref, w1_ref, b1_ref, w2_ref,
                b2_ref, w3_ref, b3_ref, y_ref, w1b, w2b, w3b):
    i = pl.program_id(0)
    nu = nu_ref[0]
    new_expert = jnp.logical_or(
        i == 0, wm_ref[i] != wm_ref[jnp.maximum(i - 1, 0)])

    @pl.when(jnp.logical_and(i < nu, new_expert))
    def _cast_weights():
        w1b[...] = w1_ref[0].astype(jnp.bfloat16)
        w2b[...] = w2_ref[0].astype(jnp.bfloat16)
        w3b[...] = w3_ref[0].astype(jnp.bfloat16)

    @pl.when(i < nu)
    def _compute():
        xa, xc = _unpack_f32(x_ref[...])
        xb = jnp.concatenate([xa, xc], axis=1).astype(jnp.bfloat16)
        h = jnp.dot(xb, w1b[...],
                    preferred_element_type=jnp.float32) + b1_ref[0]
        h = _gelu_exact(h)
        h = jnp.dot(h.astype(jnp.bfloat16), w2b[...],
                    preferred_element_type=jnp.float32) + b2_ref[0]
        h = _gelu_exact(h)
        y = jnp.dot(h.astype(jnp.bfloat16), w3b[...],
                    preferred_element_type=jnp.float32) + b3_ref[0]
        y_ref[...] = _pack_bf16(y[:, :D2], y[:, D2:])


def _ffn(rm, wm, nu, x_disp, w1, b1, w2, b2, w3, b3):
    grid_spec = pltpu.PrefetchScalarGridSpec(
        num_scalar_prefetch=3,
        grid=(NB,),
        in_specs=[
            pl.BlockSpec((TB, D2), lambda i, rm, wm, nu: (rm[i], 0)),
            pl.BlockSpec((1, D, H), lambda i, rm, wm, nu: (wm[i], 0, 0)),
            pl.BlockSpec((1, 1, H), lambda i, rm, wm, nu: (wm[i], 0, 0)),
            pl.BlockSpec((1, H, H), lambda i, rm, wm, nu: (wm[i], 0, 0)),
            pl.BlockSpec((1, 1, H), lambda i, rm, wm, nu: (wm[i], 0, 0)),
            pl.BlockSpec((1, H, O), lambda i, rm, wm, nu: (wm[i], 0, 0)),
            pl.BlockSpec((1, 1, O), lambda i, rm, wm, nu: (wm[i], 0, 0)),
        ],
        out_specs=pl.BlockSpec((TB, D2), lambda i, rm, wm, nu: (rm[i], 0)),
        scratch_shapes=[
            pltpu.VMEM((D, H), jnp.bfloat16),
            pltpu.VMEM((H, H), jnp.bfloat16),
            pltpu.VMEM((H, O), jnp.bfloat16),
        ],
    )
    return pl.pallas_call(
        _ffn_kernel,
        grid_spec=grid_spec,
        out_shape=jax.ShapeDtypeStruct((NROWS, D2), jnp.int32),
    )(rm, wm, nu, x_disp, w1, b1.reshape(E, 1, H), w2, b2.reshape(E, 1, H),
      w3, b3.reshape(E, 1, O))


def _collect_body(yd_hbm, r1_hbm, r2_hbm, ya_hbm, yb_hbm,
                  y1loc, y2loc, r1v, r2v, sem1, sem2):
    wid = lax.axis_index("s") * 2 + lax.axis_index("c")
    for c in range(TPW // CH):
        base = wid * TPW + c * CH
        pltpu.sync_copy(r1_hbm.at[pl.ds(base, CH)], r1v)
        pltpu.sync_copy(r2_hbm.at[pl.ds(base, CH)], r2v)
        cp1 = pltpu.async_copy(yd_hbm.at[r1v], y1loc, sem1)
        cp2 = pltpu.async_copy(yd_hbm.at[r2v], y2loc, sem2)
        cp1.wait()
        pltpu.sync_copy(y1loc, ya_hbm.at[pl.ds(base, CH)])
        cp2.wait()
        pltpu.sync_copy(y2loc, yb_hbm.at[pl.ds(base, CH)])


def _collect(y_disp, r1, r2):
    mesh = plsc.VectorSubcoreMesh(core_axis_name="c", subcore_axis_name="s", num_cores=2, num_subcores=16)
    f = functools.partial(
        pl.kernel, mesh=mesh,
        out_type=[
            jax.ShapeDtypeStruct((N_TOK, D2), jnp.int32),
            jax.ShapeDtypeStruct((N_TOK, D2), jnp.int32),
        ],
        scratch_types=[
            pltpu.VMEM((CH, D2), jnp.int32),
            pltpu.VMEM((CH, D2), jnp.int32),
            pltpu.VMEM((CH,), jnp.int32),
            pltpu.VMEM((CH,), jnp.int32),
            pltpu.SemaphoreType.DMA,
            pltpu.SemaphoreType.DMA,
        ],
    )(_collect_body)
    return f(y_disp, r1, r2)


def _combine_kernel(ya_ref, yb_ref, g1_ref, g2_ref, ln_g_ref, ln_b_ref,
                    out_ref):
    ya1, ya2 = _unpack_f32(ya_ref[...])
    yaf = jnp.concatenate([ya1, ya2], axis=1)
    yb1, yb2 = _unpack_f32(yb_ref[...])
    ybf = jnp.concatenate([yb1, yb2], axis=1)
    a = g1_ref[...] * yaf + g2_ref[...] * ybf
    mu = jnp.mean(a, axis=1, keepdims=True)
    var = jnp.mean((a - mu) ** 2, axis=1, keepdims=True)
    out_ref[...] = ((a - mu) * jax.lax.rsqrt(var + 1e-5)
                    * ln_g_ref[...] + ln_b_ref[...])


def _combine(ya, yb, g1, g2, ln_g, ln_b):
    return pl.pallas_call(
        _combine_kernel,
        grid=(NTB,),
        in_specs=[
            pl.BlockSpec((TB, D2), lambda tb: (tb, 0)),
            pl.BlockSpec((TB, D2), lambda tb: (tb, 0)),
            pl.BlockSpec((TB, 1), lambda tb: (tb, 0)),
            pl.BlockSpec((TB, 1), lambda tb: (tb, 0)),
            pl.BlockSpec((1, O), lambda tb: (0, 0)),
            pl.BlockSpec((1, O), lambda tb: (0, 0)),
        ],
        out_specs=pl.BlockSpec((TB, O), lambda tb: (tb, 0)),
        out_shape=jax.ShapeDtypeStruct((N_TOK, O), jnp.float32),
    )(ya, yb, g1, g2, ln_g.reshape(1, O), ln_b.reshape(1, O))


@jax.jit
def kernel(x, gate_w, w1, b1, w2, b2, w3, b3, ln_g, ln_b):
    b, s, d = x.shape
    x2 = x.reshape(b * s, d)

    xp, g1, g2, r1, r2, rowmap, wmap, nu, aux = _router(x2, gate_w)

    r1 = r1.reshape(-1)
    r2 = r2.reshape(-1)
    x_disp = _dispatch(xp, r1, r2)
    y_disp = _ffn(rowmap.reshape(-1), wmap.reshape(-1), nu.reshape(-1),
                  x_disp, w1, b1, w2, b2, w3, b3)
    ya, yb = _collect(y_disp, r1, r2)
    out2 = _combine(ya, yb, g1, g2, ln_g, ln_b)
    return out2.reshape(b, s, O), aux[0, 0]


# speedup vs baseline: 5.6204x; 1.0280x over previous
import functools

import jax
import jax.numpy as jnp
from jax import lax
from jax.experimental import pallas as pl
from jax.experimental.pallas import tpu as pltpu
from jax.experimental.pallas import tpu_sc as plsc

E = 8
TOP_K = 2
D = 768
H = 768
O = 768
N_TOK = 4096
TB = 512
NTB = N_TOK // TB
NB = N_TOK * TOP_K // TB + E
CAPB = N_TOK // TB
NROWS = (E * CAPB + 1) * TB

NW = 32
TPW = N_TOK // NW
CH = 128


def _gelu_exact(h):
    return 0.5 * h * (1.0 + jax.lax.erf(h * 0.7071067811865476))


D2 = D // 2


def _pack_bf16(a, b):
    au = jax.lax.bitcast_convert_type(a, jnp.uint32) + 0x8000
    bu = jax.lax.bitcast_convert_type(b, jnp.uint32) + 0x8000
    pu = (au & jnp.uint32(0xFFFF0000)) | (bu >> 16)
    return jax.lax.bitcast_convert_type(pu, jnp.int32)


def _unpack_f32(p):
    pu = jax.lax.bitcast_convert_type(p, jnp.uint32)
    a = jax.lax.bitcast_convert_type(pu & jnp.uint32(0xFFFF0000), jnp.float32)
    b = jax.lax.bitcast_convert_type(pu << 16, jnp.float32)
    return a, b


def _router_kernel(x_ref, gw_ref,
                   xp_ref, g1_ref, g2_ref, r1_ref, r2_ref, rowmap_ref,
                   wmap_ref, nu_ref, aux_ref,
                   cnt_s, use_s, lt_s):
    tb = pl.program_id(0)

    @pl.when(tb == 0)
    def _init():
        cnt_s[...] = jnp.zeros((1, E), jnp.float32)
        use_s[...] = jnp.zeros((1, E), jnp.float32)
        row = jax.lax.broadcasted_iota(jnp.int32, (TB, TB), 0)
        col = jax.lax.broadcasted_iota(jnp.int32, (TB, TB), 1)
        lt_s[...] = (col < row).astype(jnp.bfloat16)

    @pl.when(tb < NTB)
    def _block():
        xb = x_ref[...]
        xp_ref[...] = _pack_bf16(xb[:, :D2], xb[:, D2:])
        logits = jnp.dot(xb, gw_ref[...], preferred_element_type=jnp.float32)
        lane = jax.lax.broadcasted_iota(jnp.int32, (TB, E), 1)
        v1 = jnp.max(logits, axis=1, keepdims=True)
        i1 = jnp.argmax(logits, axis=1)[:, None]
        masked = jnp.where(lane == i1, -jnp.inf, logits)
        v2 = jnp.max(masked, axis=1, keepdims=True)
        i2 = jnp.argmax(masked, axis=1)[:, None]
        g1 = 1.0 / (1.0 + jnp.exp(v2 - v1))
        g2 = 1.0 - g1

        oh1 = (lane == i1).astype(jnp.float32)
        oh2 = (lane == i2).astype(jnp.float32)
        ohs = oh1 + oh2
        excl = jnp.dot(lt_s[...], ohs.astype(jnp.bfloat16),
                       preferred_element_type=jnp.float32)
        cnt = cnt_s[...]
        rank1 = jnp.sum(oh1 * (cnt + excl), axis=1, keepdims=True)
        rank2 = jnp.sum(oh2 * (cnt + excl + oh1), axis=1, keepdims=True)
        cnt_s[...] = cnt + jnp.sum(ohs, axis=0, keepdims=True)
        use_s[...] = use_s[...] + jnp.sum(oh1 * g1 + oh2 * g2, axis=0,
                                          keepdims=True)

        r1_ref[...] = i1 * N_TOK + rank1.astype(jnp.int32)
        r2_ref[...] = i2 * N_TOK + rank2.astype(jnp.int32)
        g1_ref[...] = g1
        g2_ref[...] = g2

    @pl.when(tb == NTB)
    def _fin():
        cnt = cnt_s[...]
        usage = use_s[...] / N_TOK
        uniform = 1.0 / E
        aux_ref[...] = jnp.sum(uniform * (jnp.log(uniform) - jnp.log(usage)),
                               axis=1, keepdims=True)
        nb_e = jnp.floor((cnt + (TB - 1)) * (1.0 / TB))
        r8 = jax.lax.broadcasted_iota(jnp.int32, (E, E), 0)
        c8 = jax.lax.broadcasted_iota(jnp.int32, (E, E), 1)
        le = (r8 <= c8).astype(jnp.float32)
        cum = jnp.dot(nb_e, le, preferred_element_type=jnp.float32)
        iota_nb = jax.lax.broadcasted_iota(
            jnp.int32, (1, NB), 1).astype(jnp.float32)
        e_sel = jnp.zeros((1, NB), jnp.float32)
        cexcl_sel = jnp.zeros((1, NB), jnp.float32)
        for e in range(E):
            cum_e = cum[0:1, e:e + 1]
            e_sel = e_sel + (iota_nb >= cum_e).astype(jnp.float32)
        for e in range(E):
            cexcl_e = cum[0:1, e:e + 1] - nb_e[0:1, e:e + 1]
            e_eq = (e_sel == e).astype(jnp.float32)
            cexcl_sel = cexcl_sel + e_eq * cexcl_e
        nu = cum[0:1, E - 1:E]
        live = iota_nb < nu
        boff = iota_nb - cexcl_sel
        rowmap = jnp.where(live, e_sel * CAPB + boff,
                           float(E * CAPB)).astype(jnp.int32)
        wmap = jnp.minimum(e_sel, E - 1).astype(jnp.int32)
        rowmap_ref[...] = rowmap
        wmap_ref[...] = wmap
        nu_ref[...] = nu.astype(jnp.int32)


def _router(x2, gate_w):
    return pl.pallas_call(
        _router_kernel,
        grid=(NTB + 1,),
        in_specs=[
            pl.BlockSpec((TB, D), lambda tb: (jnp.minimum(tb, NTB - 1), 0)),
            pl.BlockSpec((D, E), lambda tb: (0, 0)),
        ],
        out_specs=[
            pl.BlockSpec((TB, D2), lambda tb: (jnp.minimum(tb, NTB - 1), 0)),
            pl.BlockSpec((TB, 1), lambda tb: (jnp.minimum(tb, NTB - 1), 0)),
            pl.BlockSpec((TB, 1), lambda tb: (jnp.minimum(tb, NTB - 1), 0)),
            pl.BlockSpec((TB, 1), lambda tb: (jnp.minimum(tb, NTB - 1), 0)),
            pl.BlockSpec((TB, 1), lambda tb: (jnp.minimum(tb, NTB - 1), 0)),
            pl.BlockSpec((1, NB), lambda tb: (0, 0)),
            pl.BlockSpec((1, NB), lambda tb: (0, 0)),
            pl.BlockSpec((1, 1), lambda tb: (0, 0)),
            pl.BlockSpec((1, 1), lambda tb: (0, 0)),
        ],
        out_shape=[
            jax.ShapeDtypeStruct((N_TOK, D2), jnp.int32),
            jax.ShapeDtypeStruct((N_TOK, 1), jnp.float32),
            jax.ShapeDtypeStruct((N_TOK, 1), jnp.float32),
            jax.ShapeDtypeStruct((N_TOK, 1), jnp.int32),
            jax.ShapeDtypeStruct((N_TOK, 1), jnp.int32),
            jax.ShapeDtypeStruct((1, NB), jnp.int32),
            jax.ShapeDtypeStruct((1, NB), jnp.int32),
            jax.ShapeDtypeStruct((1, 1), jnp.int32),
            jax.ShapeDtypeStruct((1, 1), jnp.float32),
        ],
        scratch_shapes=[
            pltpu.VMEM((1, E), jnp.float32),
            pltpu.VMEM((1, E), jnp.float32),
            pltpu.VMEM((TB, TB), jnp.bfloat16),
        ],
    )(x2, gate_w)


def _dispatch_body(x_hbm, r1_hbm, r2_hbm, xd_hbm,
                   xloc, r1v, r2v, sem1, sem2):
    wid = lax.axis_index("s") * 2 + lax.axis_index("c")
    for c in range(TPW // CH):
        base = wid * TPW + c * CH
        pltpu.sync_copy(x_hbm.at[pl.ds(base, CH)], xloc)
        pltpu.sync_copy(r1_hbm.at[pl.ds(base, CH)], r1v)
        pltpu.sync_copy(r2_hbm.at[pl.ds(base, CH)], r2v)
        cp1 = pltpu.async_copy(xloc, xd_hbm.at[r1v], sem1)
        cp2 = pltpu.async_copy(xloc, xd_hbm.at[r2v], sem2)
        cp1.wait()
        cp2.wait()


def _dispatch(xp, r1, r2):
    mesh = plsc.VectorSubcoreMesh(core_axis_name="c", subcore_axis_name="s", num_cores=2, num_subcores=16)
    f = functools.partial(
        pl.kernel, mesh=mesh,
        out_type=jax.ShapeDtypeStruct((NROWS, D2), jnp.int32),
        scratch_types=[
            pltpu.VMEM((CH, D2), jnp.int32),
            pltpu.VMEM((CH,), jnp.int32),
            pltpu.VMEM((CH,), jnp.int32),
            pltpu.SemaphoreType.DMA,
            pltpu.SemaphoreType.DMA,
        ],
    )(_dispatch_body)
    return f(xp, r1, r2)


def _ffn_kernel(rm_ref, wm_ref, nu_ref, x_ref, w1_ref, b1_ref, w2_ref,
                b2_ref, w3_ref, b3_ref, y_ref, w1b, w2b, w3b):
    i = pl.program_id(0)
    nu = nu_ref[0]
    new_expert = jnp.logical_or(
        i == 0, wm_ref[i] != wm_ref[jnp.maximum(i - 1, 0)])

    @pl.when(jnp.logical_and(i < nu, new_expert))
    def _cast_weights():
        w1b[...] = w1_ref[0].astype(jnp.bfloat16)
        w2b[...] = w2_ref[0].astype(jnp.bfloat16)
        w3b[...] = w3_ref[0].astype(jnp.bfloat16)

    @pl.when(i < nu)
    def _compute():
        xa, xc = _unpack_f32(x_ref[...])
        xb = jnp.concatenate([xa, xc], axis=1).astype(jnp.bfloat16)
        h = jnp.dot(xb, w1b[...],
                    preferred_element_type=jnp.float32) + b1_ref[0]
        h = _gelu_exact(h)
        h = jnp.dot(h.astype(jnp.bfloat16), w2b[...],
                    preferred_element_type=jnp.float32) + b2_ref[0]
        h = _gelu_exact(h)
        y = jnp.dot(h.astype(jnp.bfloat16), w3b[...],
                    preferred_element_type=jnp.float32) + b3_ref[0]
        y_ref[...] = _pack_bf16(y[:, :D2], y[:, D2:])


def _ffn(rm, wm, nu, x_disp, w1, b1, w2, b2, w3, b3):
    grid_spec = pltpu.PrefetchScalarGridSpec(
        num_scalar_prefetch=3,
        grid=(NB,),
        in_specs=[
            pl.BlockSpec((TB, D2), lambda i, rm, wm, nu: (rm[i], 0)),
            pl.BlockSpec((1, D, H), lambda i, rm, wm, nu: (wm[i], 0, 0)),
            pl.BlockSpec((1, 1, H), lambda i, rm, wm, nu: (wm[i], 0, 0)),
            pl.BlockSpec((1, H, H), lambda i, rm, wm, nu: (wm[i], 0, 0)),
            pl.BlockSpec((1, 1, H), lambda i, rm, wm, nu: (wm[i], 0, 0)),
            pl.BlockSpec((1, H, O), lambda i, rm, wm, nu: (wm[i], 0, 0)),
            pl.BlockSpec((1, 1, O), lambda i, rm, wm, nu: (wm[i], 0, 0)),
        ],
        out_specs=pl.BlockSpec((TB, D2), lambda i, rm, wm, nu: (rm[i], 0)),
        scratch_shapes=[
            pltpu.VMEM((D, H), jnp.bfloat16),
            pltpu.VMEM((H, H), jnp.bfloat16),
            pltpu.VMEM((H, O), jnp.bfloat16),
        ],
    )
    return pl.pallas_call(
        _ffn_kernel,
        grid_spec=grid_spec,
        out_shape=jax.ShapeDtypeStruct((NROWS, D2), jnp.int32),
    )(rm, wm, nu, x_disp, w1, b1.reshape(E, 1, H), w2, b2.reshape(E, 1, H),
      w3, b3.reshape(E, 1, O))


def _collect_body(yd_hbm, r1_hbm, r2_hbm, ya_hbm, yb_hbm,
                  y1loc, y2loc, r1v, r2v, sem1, sem2):
    wid = lax.axis_index("s") * 2 + lax.axis_index("c")
    for c in range(TPW // CH):
        base = wid * TPW + c * CH
        pltpu.sync_copy(r1_hbm.at[pl.ds(base, CH)], r1v)
        pltpu.sync_copy(r2_hbm.at[pl.ds(base, CH)], r2v)
        cp1 = pltpu.async_copy(yd_hbm.at[r1v], y1loc, sem1)
        cp2 = pltpu.async_copy(yd_hbm.at[r2v], y2loc, sem2)
        cp1.wait()
        pltpu.sync_copy(y1loc, ya_hbm.at[pl.ds(base, CH)])
        cp2.wait()
        pltpu.sync_copy(y2loc, yb_hbm.at[pl.ds(base, CH)])


def _collect(y_disp, r1, r2):
    mesh = plsc.VectorSubcoreMesh(core_axis_name="c", subcore_axis_name="s", num_cores=2, num_subcores=16)
    f = functools.partial(
        pl.kernel, mesh=mesh,
        out_type=[
            jax.ShapeDtypeStruct((N_TOK, D2), jnp.int32),
            jax.ShapeDtypeStruct((N_TOK, D2), jnp.int32),
        ],
        scratch_types=[
            pltpu.VMEM((CH, D2), jnp.int32),
            pltpu.VMEM((CH, D2), jnp.int32),
            pltpu.VMEM((CH,), jnp.int32),
            pltpu.VMEM((CH,), jnp.int32),
            pltpu.SemaphoreType.DMA,
            pltpu.SemaphoreType.DMA,
        ],
    )(_collect_body)
    return f(y_disp, r1, r2)


def _combine_kernel(ya_ref, yb_ref, g1_ref, g2_ref, ln_g_ref, ln_b_ref,
                    out_ref):
    ya1, ya2 = _unpack_f32(ya_ref[...])
    yaf = jnp.concatenate([ya1, ya2], axis=1)
    yb1, yb2 = _unpack_f32(yb_ref[...])
    ybf = jnp.concatenate([yb1, yb2], axis=1)
    a = g1_ref[...] * yaf + g2_ref[...] * ybf
    mu = jnp.mean(a, axis=1, keepdims=True)
    var = jnp.mean((a - mu) ** 2, axis=1, keepdims=True)
    out_ref[...] = ((a - mu) * jax.lax.rsqrt(var + 1e-5)
                    * ln_g_ref[...] + ln_b_ref[...])


def _combine(ya, yb, g1, g2, ln_g, ln_b):
    return pl.pallas_call(
        _combine_kernel,
        grid=(NTB,),
        in_specs=[
            pl.BlockSpec((TB, D2), lambda tb: (tb, 0)),
            pl.BlockSpec((TB, D2), lambda tb: (tb, 0)),
            pl.BlockSpec((TB, 1), lambda tb: (tb, 0)),
            pl.BlockSpec((TB, 1), lambda tb: (tb, 0)),
            pl.BlockSpec((1, O), lambda tb: (0, 0)),
            pl.BlockSpec((1, O), lambda tb: (0, 0)),
        ],
        out_specs=pl.BlockSpec((TB, O), lambda tb: (tb, 0)),
        out_shape=jax.ShapeDtypeStruct((N_TOK, O), jnp.float32),
    )(ya, yb, g1, g2, ln_g.reshape(1, O), ln_b.reshape(1, O))


@jax.jit
def kernel(x, gate_w, w1, b1, w2, b2, w3, b3, ln_g, ln_b):
    b, s, d = x.shape
    x2 = x.reshape(b * s, d)

    xp, g1, g2, r1, r2, rowmap, wmap, nu, aux = _router(x2, gate_w)

    r1 = r1.reshape(-1)
    r2 = r2.reshape(-1)
    x_disp = _dispatch(xp, r1, r2)
    y_disp = _ffn(rowmap.reshape(-1), wmap.reshape(-1), nu.reshape(-1),
                  x_disp, w1, b1, w2, b2, w3, b3)
    ya, yb = _collect(y_disp, r1, r2)
    out2 = _combine(ya, yb, g1, g2, ln_g, ln_b)
    return out2.reshape(b, s, O), aux[0, 0]
